# baseline (speedup 1.0000x reference)
import functools

import jax
import jax.numpy as jnp
from jax import lax
from jax.experimental import pallas as pl
from jax.experimental.pallas import tpu as pltpu
from jax.experimental.pallas import tpu_sc as plsc

NC = 2
NS = 16
LANES = 16


def _proj_body(x_ref, wdf_ref, wds_ref, wsf_ref, wss_ref, bf_ref, bs_ref,
               df_ref, ds_ref, sf_ref, ss_ref):
    xb = x_ref[...].astype(jnp.bfloat16)

    def mm(w_ref):
        return jnp.dot(xb, w_ref[...].astype(jnp.bfloat16),
                       preferred_element_type=jnp.float32)

    df_ref[...] = mm(wdf_ref) + bf_ref[...]
    ds_ref[...] = mm(wds_ref) + bs_ref[...]
    sf_ref[...] = mm(wsf_ref)
    ss_ref[...] = mm(wss_ref)


def _projections(x, W_Df, W_Ds, W_Sf, W_Ss, b_f, b_s, blk):
    n, f = x.shape
    grid = n // blk
    wspec = pl.BlockSpec((f, f), lambda i: (0, 0))
    bspec = pl.BlockSpec((1, f), lambda i: (0, 0))
    ospec = pl.BlockSpec((blk, f), lambda i: (i, 0))
    oshape = jax.ShapeDtypeStruct((n, f), jnp.float32)
    return pl.pallas_call(
        _proj_body,
        grid=(grid,),
        in_specs=[pl.BlockSpec((blk, f), lambda i: (i, 0)),
                  wspec, wspec, wspec, wspec, bspec, bspec],
        out_specs=[ospec, ospec, ospec, ospec],
        out_shape=[oshape, oshape, oshape, oshape],
    )(x, W_Df, W_Ds, W_Sf, W_Ss, b_f, b_s)


_LP = (0.9999923, 0.576651, 0.01735374)
_LQ = (1.0, 1.0765147, 0.22313626)


def _gate16(gf, gs):
    ef = jnp.exp(-gf)
    t = jnp.exp(-jnp.abs(gs))
    p = (_LP[2] * t + _LP[1]) * t + _LP[0]
    q = (_LQ[2] * t + _LQ[1]) * t + _LQ[0]
    num = q * jnp.maximum(gs, 0.0) + t * p
    return num / ((1.0 + ef) * q)


def _edge_phase(ei, Df, Ds, Sf, Ss, zrows, n, e, chunk):
    fm = Df.shape[1]
    nw = NC * NS
    epw = e // nw
    nchunk = epw // chunk
    stripe = zrows.shape[0]
    last_rows = n - (NS - 1) * stripe
    nvec = fm // LANES

    mesh = plsc.VectorSubcoreMesh(core_axis_name="c", subcore_axis_name="s")

    buf = lambda: pltpu.VMEM((chunk, fm), jnp.float32)
    ibuf = lambda: pltpu.VMEM((chunk,), jnp.int32)
    assert nchunk >= 8 and nchunk % 4 == 2

    @functools.partial(
        pl.kernel,
        mesh=mesh,
        out_type=jax.ShapeDtypeStruct((NC, n, fm), jnp.float32),
        scratch_types=[
            ibuf(), ibuf(), ibuf(), ibuf(),
            ibuf(), ibuf(), ibuf(), ibuf(),
            buf(), buf(), buf(), buf(),
            buf(), buf(), buf(), buf(),
            pltpu.VMEM_SHARED((n, fm), jnp.float32),
            pltpu.SemaphoreType.DMA, pltpu.SemaphoreType.DMA,
            pltpu.SemaphoreType.DMA, pltpu.SemaphoreType.DMA,
            pltpu.SemaphoreType.DMA, pltpu.SemaphoreType.DMA,
            pltpu.SemaphoreType.DMA, pltpu.SemaphoreType.DMA,
        ],
    )
    def edge_kernel(src_hbm, dst_hbm, df_hbm, ds_hbm, sf_hbm, ss_hbm, z_hbm,
                    out_hbm,
                    sb0, sb1, sb2, sb3, db0, db1, db2, db3,
                    dfA, dsA, sfA, ssA, dfB, dsB, sfB, ssB,
                    agg_sh, sgA, sgB, scA, scB, si0, si1, si2, si3):
        cid = lax.axis_index("c")
        sid = lax.axis_index("s")
        wid = sid * NC + cid
        base = wid * epw

        row0 = sid * stripe

        @pl.when(sid < NS - 1)
        def _zero_main():
            pltpu.sync_copy(z_hbm, agg_sh.at[pl.ds(row0, stripe)])

        @pl.when(sid == NS - 1)
        def _zero_last():
            pltpu.sync_copy(z_hbm.at[pl.ds(0, last_rows)],
                            agg_sh.at[pl.ds(row0, last_rows)])

        plsc.subcore_barrier()

        sbufs = (sb0, sb1, sb2, sb3)
        dbufs = (db0, db1, db2, db3)
        isems = (si0, si1, si2, si3)
        sets = ((dfA, dsA, sfA, ssA, sgA, scA),
                (dfB, dsB, sfB, ssB, sgB, scB))

        def issue_idx(k, b):
            off = pl.multiple_of(base + k * chunk, 8)
            pltpu.async_copy(src_hbm.at[pl.ds(off, chunk)], sbufs[b], isems[b])
            pltpu.async_copy(dst_hbm.at[pl.ds(off, chunk)], dbufs[b], isems[b])

        def wait_idx(b):
            pltpu.make_async_copy(src_hbm.at[pl.ds(0, chunk)],
                                  sbufs[b], isems[b]).wait()
            pltpu.make_async_copy(dst_hbm.at[pl.ds(0, chunk)],
                                  dbufs[b], isems[b]).wait()

        def issue_gathers(s, b):
            sidx, didx = sbufs[b], dbufs[b]
            pltpu.async_copy(df_hbm.at[didx], s[0], s[4])
            pltpu.async_copy(ds_hbm.at[didx], s[1], s[4])
            pltpu.async_copy(sf_hbm.at[sidx], s[2], s[4])
            pltpu.async_copy(ss_hbm.at[sidx], s[3], s[4])

        def wait_gathers(s):
            for b in (s[0], s[1], s[2], s[3]):
                pltpu.make_async_copy(df_hbm.at[pl.ds(0, chunk)], b, s[4]).wait()

        def wait_scatter(s):
            pltpu.make_async_copy(df_hbm.at[pl.ds(0, chunk)], s[0], s[5]).wait()

        def compute(s):
            dfb, dsb, sfb, ssb = s[0], s[1], s[2], s[3]

            def edge_body(r, c2):
                for j in range(nvec):
                    lo = j * LANES
                    gf = dfb[r, pl.ds(lo, LANES)] + sfb[r, pl.ds(lo, LANES)]
                    gs = dsb[r, pl.ds(lo, LANES)] + ssb[r, pl.ds(lo, LANES)]
                    dfb[r, pl.ds(lo, LANES)] = _gate16(gf, gs)
                return c2

            lax.fori_loop(0, chunk, edge_body, 0)

        def issue_scatter(s, b):
            pltpu.async_copy(s[0], agg_sh.at[dbufs[b]], s[5], add=True)

        def phase(k, m4, first=False, rows=True, idx=True):
            cur, nxt = sets[m4 % 2], sets[(m4 + 1) % 2]
            wait_gathers(cur)
            if rows:
                wait_idx((m4 + 1) % 4)
                if not first:
                    wait_scatter(nxt)
                issue_gathers(nxt, (m4 + 1) % 4)
            compute(cur)
            issue_scatter(cur, m4)
            if idx:
                issue_idx(k + 2, (m4 + 2) % 4)

        issue_idx(0, 0)
        wait_idx(0)
        issue_gathers(sets[0], 0)
        issue_idx(1, 1)
        phase(0, 0, first=True)
        phase(1, 1)

        def quad_body(i, carry):
            k = 4 * i + 2
            phase(k, 2)
            phase(k + 1, 3)
            phase(k + 2, 0)
            phase(k + 3, 1)
            return carry

        lax.fori_loop(0, (nchunk - 6) // 4, quad_body, 0)
        phase(nchunk - 4, 2)
        phase(nchunk - 3, 3)
        phase(nchunk - 2, 0, idx=False)
        phase(nchunk - 1, 1, rows=False, idx=False)
        wait_scatter(sets[0])
        wait_scatter(sets[1])
        plsc.subcore_barrier()

        @pl.when(sid < NS - 1)
        def _out_main():
            pltpu.sync_copy(agg_sh.at[pl.ds(row0, stripe)],
                            out_hbm.at[cid, pl.ds(row0, stripe)])

        @pl.when(sid == NS - 1)
        def _out_last():
            pltpu.sync_copy(agg_sh.at[pl.ds(row0, last_rows)],
                            out_hbm.at[cid, pl.ds(row0, last_rows)])

    return edge_kernel(ei[0], ei[1], Df, Ds, Sf, Ss, zrows)


def _fuse_body(x_ref, a0_ref, a1_ref, surf_ref, wsl_ref, bsl_ref, wl_ref,
               num_ref, ss_ref, csl):
    @pl.when(pl.program_id(0) == 0)
    def _init():
        num_ref[...] = jnp.zeros_like(num_ref)
        ss_ref[...] = jnp.zeros_like(ss_ref)

    xb = x_ref[...]
    agg = a0_ref[...] + a1_ref[...]
    cad = xb.shape[1] - csl
    x_sl = xb[:, :csl] + agg[:, cad:]
    x_ad = xb[:, csl:] + agg[:, :cad]
    t = jnp.dot(x_sl.astype(jnp.bfloat16), wsl_ref[...].astype(jnp.bfloat16),
                preferred_element_type=jnp.float32) + bsl_ref[...]
    h = t * x_ad * surf_ref[...]
    h16 = h.astype(jnp.bfloat16).astype(jnp.float32)
    wl16 = wl_ref[...].astype(jnp.bfloat16).astype(jnp.float32)
    num_ref[...] += jnp.sum(h16 * wl16)
    ss_ref[...] += jnp.sum(surf_ref[...])


def _fuse(x, agg0, agg1, surf, W_lin_sl, b_lin_sl, wl_row, blk, csl):
    n, f = x.shape
    cad = f - csl
    grid = n // blk
    return pl.pallas_call(
        functools.partial(_fuse_body, csl=csl),
        grid=(grid,),
        in_specs=[
            pl.BlockSpec((blk, f), lambda i: (i, 0)),
            pl.BlockSpec((blk, f), lambda i: (i, 0)),
            pl.BlockSpec((blk, f), lambda i: (i, 0)),
            pl.BlockSpec((blk, 1), lambda i: (i, 0)),
            pl.BlockSpec((csl, cad), lambda i: (0, 0)),
            pl.BlockSpec((1, cad), lambda i: (0, 0)),
            pl.BlockSpec((1, cad), lambda i: (0, 0)),
        ],
        out_specs=[
            pl.BlockSpec((1, 1), lambda i: (0, 0)),
            pl.BlockSpec((1, 1), lambda i: (0, 0)),
        ],
        out_shape=[
            jax.ShapeDtypeStruct((1, 1), jnp.float32),
            jax.ShapeDtypeStruct((1, 1), jnp.float32),
        ],
    )(x, agg0, agg1, surf, W_lin_sl, b_lin_sl, wl_row)


def kernel(x, edge_index, surf_filter,
           Wf_sl, bf_sl, Ws_sl, bs_sl,
           Wf_ad, bf_ad, Ws_ad, bs_ad,
           W_lin_sl, b_lin_sl, W_lin, b_lin):
    n, f = x.shape
    e = edge_index.shape[1]
    csl = Wf_sl.shape[1]
    cad = Wf_ad.shape[1]

    z_sa = jnp.zeros((csl, cad), jnp.float32)
    z_as = jnp.zeros((cad, csl), jnp.float32)

    def pack(w_ad, w_sl):
        return jnp.concatenate([
            jnp.concatenate([z_sa, w_sl], axis=1),
            jnp.concatenate([w_ad, z_as], axis=1),
        ], axis=0)

    W_Df = pack(Wf_ad[:cad], Wf_sl[:csl])
    W_Ds = pack(Ws_ad[:cad], Ws_sl[:csl])
    W_Sf = pack(Wf_ad[cad:], Wf_sl[csl:])
    W_Ss = pack(Ws_ad[cad:], Ws_sl[csl:])
    b_f = jnp.concatenate([bf_ad, bf_sl]).reshape(1, -1)
    b_s = jnp.concatenate([bs_ad, bs_sl]).reshape(1, -1)

    Df, Ds, Sf, Ss = _projections(x, W_Df, W_Ds, W_Sf, W_Ss, b_f, b_s, blk=2000)

    zrows = jnp.zeros((640, f), jnp.float32)
    agg = _edge_phase(edge_index, Df, Ds, Sf, Ss, zrows, n, e, chunk=40)

    num, ss = _fuse(x, agg[0], agg[1], surf_filter.reshape(-1, 1),
                    W_lin_sl, b_lin_sl.reshape(1, -1), W_lin.reshape(1, -1),
                    blk=2000, csl=csl)
    return (num[0, 0] + n * b_lin[0]) / ss[0, 0]


# speedup vs baseline: 1.0661x; 1.0661x over previous
import functools

import jax
import jax.numpy as jnp
from jax import lax
from jax.experimental import pallas as pl
from jax.experimental.pallas import tpu as pltpu
from jax.experimental.pallas import tpu_sc as plsc

NC = 2
NS = 16
LANES = 16


def _proj_body(x_ref, wdf_ref, wds_ref, wsf_ref, wss_ref, bf_ref, bs_ref,
               df_ref, ds_ref, sf_ref, ss_ref):
    xb = x_ref[...].astype(jnp.bfloat16)

    def mm(w_ref):
        return jnp.dot(xb, w_ref[...].astype(jnp.bfloat16),
                       preferred_element_type=jnp.float32)

    df_ref[...] = mm(wdf_ref) + bf_ref[...]
    ds_ref[...] = mm(wds_ref) + bs_ref[...]
    sf_ref[...] = mm(wsf_ref)
    ss_ref[...] = mm(wss_ref)


def _projections(x, W_Df, W_Ds, W_Sf, W_Ss, b_f, b_s, blk):
    n, f = x.shape
    grid = n // blk
    wspec = pl.BlockSpec((f, f), lambda i: (0, 0))
    bspec = pl.BlockSpec((1, f), lambda i: (0, 0))
    ospec = pl.BlockSpec((blk, f), lambda i: (i, 0))
    oshape = jax.ShapeDtypeStruct((n, f), jnp.float32)
    return pl.pallas_call(
        _proj_body,
        grid=(grid,),
        in_specs=[pl.BlockSpec((blk, f), lambda i: (i, 0)),
                  wspec, wspec, wspec, wspec, bspec, bspec],
        out_specs=[ospec, ospec, ospec, ospec],
        out_shape=[oshape, oshape, oshape, oshape],
    )(x, W_Df, W_Ds, W_Sf, W_Ss, b_f, b_s)


_LP = (0.9999923, 0.576651, 0.01735374)
_LQ = (1.0, 1.0765147, 0.22313626)


def _gate16(ngf, gs):
    ef = jnp.exp(ngf)
    t = jnp.exp(-jnp.abs(gs))
    p = (_LP[2] * t + _LP[1]) * t + _LP[0]
    q = (_LQ[2] * t + _LQ[1]) * t + _LQ[0]
    num = q * jnp.maximum(gs, 0.0) + t * p
    return num / ((1.0 + ef) * q)


def _edge_phase(ei, Df, Ds, Sf, Ss, zrows, n, e, chunk):
    fm = Df.shape[1]
    nw = NC * NS
    epw = e // nw
    nchunk = epw // chunk
    stripe = zrows.shape[0]
    last_rows = n - (NS - 1) * stripe
    nvec = fm // LANES

    mesh = plsc.VectorSubcoreMesh(core_axis_name="c", subcore_axis_name="s")

    buf = lambda: pltpu.VMEM((chunk, fm), jnp.float32)
    ibuf = lambda: pltpu.VMEM((chunk,), jnp.int32)
    assert nchunk >= 8 and nchunk % 4 == 2

    @functools.partial(
        pl.kernel,
        mesh=mesh,
        out_type=jax.ShapeDtypeStruct((NC, n, fm), jnp.float32),
        scratch_types=[
            ibuf(), ibuf(), ibuf(), ibuf(),
            ibuf(), ibuf(), ibuf(), ibuf(),
            buf(), buf(), buf(), buf(),
            buf(), buf(), buf(), buf(),
            pltpu.VMEM_SHARED((n, fm), jnp.float32),
            pltpu.SemaphoreType.DMA, pltpu.SemaphoreType.DMA,
            pltpu.SemaphoreType.DMA, pltpu.SemaphoreType.DMA,
            pltpu.SemaphoreType.DMA, pltpu.SemaphoreType.DMA,
            pltpu.SemaphoreType.DMA, pltpu.SemaphoreType.DMA,
        ],
    )
    def edge_kernel(src_hbm, dst_hbm, df_hbm, ds_hbm, sf_hbm, ss_hbm, z_hbm,
                    out_hbm,
                    sb0, sb1, sb2, sb3, db0, db1, db2, db3,
                    dfA, dsA, sfA, ssA, dfB, dsB, sfB, ssB,
                    agg_sh, sgA, sgB, scA, scB, si0, si1, si2, si3):
        cid = lax.axis_index("c")
        sid = lax.axis_index("s")
        wid = sid * NC + cid
        base = wid * epw

        row0 = sid * stripe

        @pl.when(sid < NS - 1)
        def _zero_main():
            pltpu.sync_copy(z_hbm, agg_sh.at[pl.ds(row0, stripe)])

        @pl.when(sid == NS - 1)
        def _zero_last():
            pltpu.sync_copy(z_hbm.at[pl.ds(0, last_rows)],
                            agg_sh.at[pl.ds(row0, last_rows)])

        plsc.subcore_barrier()

        sbufs = (sb0, sb1, sb2, sb3)
        dbufs = (db0, db1, db2, db3)
        isems = (si0, si1, si2, si3)
        sets = ((dfA, dsA, sfA, ssA, sgA, scA),
                (dfB, dsB, sfB, ssB, sgB, scB))

        def issue_idx(k, b):
            off = pl.multiple_of(base + k * chunk, 8)
            pltpu.async_copy(src_hbm.at[pl.ds(off, chunk)], sbufs[b], isems[b])
            pltpu.async_copy(dst_hbm.at[pl.ds(off, chunk)], dbufs[b], isems[b])

        def wait_idx(b):
            pltpu.make_async_copy(src_hbm.at[pl.ds(0, chunk)],
                                  sbufs[b], isems[b]).wait()
            pltpu.make_async_copy(dst_hbm.at[pl.ds(0, chunk)],
                                  dbufs[b], isems[b]).wait()

        def issue_gathers(s, b):
            sidx, didx = sbufs[b], dbufs[b]
            pltpu.async_copy(df_hbm.at[didx], s[0], s[4])
            pltpu.async_copy(ds_hbm.at[didx], s[1], s[4])
            pltpu.async_copy(sf_hbm.at[sidx], s[2], s[4])
            pltpu.async_copy(ss_hbm.at[sidx], s[3], s[4])

        def wait_gathers(s):
            for b in (s[0], s[1], s[2], s[3]):
                pltpu.make_async_copy(df_hbm.at[pl.ds(0, chunk)], b, s[4]).wait()

        def wait_scatter(s):
            pltpu.make_async_copy(df_hbm.at[pl.ds(0, chunk)], s[0], s[5]).wait()

        def compute(s):
            dfb, dsb, sfb, ssb = s[0], s[1], s[2], s[3]

            def edge_body(r, c2):
                for j in range(nvec):
                    lo = j * LANES
                    gf = dfb[r, pl.ds(lo, LANES)] + sfb[r, pl.ds(lo, LANES)]
                    gs = dsb[r, pl.ds(lo, LANES)] + ssb[r, pl.ds(lo, LANES)]
                    dfb[r, pl.ds(lo, LANES)] = _gate16(gf, gs)
                return c2

            lax.fori_loop(0, chunk, edge_body, 0)

        def issue_scatter(s, b):
            pltpu.async_copy(s[0], agg_sh.at[dbufs[b]], s[5], add=True)

        def phase(k, m4, first=False, rows=True, idx=True):
            cur, nxt = sets[m4 % 2], sets[(m4 + 1) % 2]
            wait_gathers(cur)
            if rows:
                wait_idx((m4 + 1) % 4)
                if not first:
                    wait_scatter(nxt)
                issue_gathers(nxt, (m4 + 1) % 4)
            compute(cur)
            issue_scatter(cur, m4)
            if idx:
                issue_idx(k + 2, (m4 + 2) % 4)

        issue_idx(0, 0)
        wait_idx(0)
        issue_gathers(sets[0], 0)
        issue_idx(1, 1)
        phase(0, 0, first=True)
        phase(1, 1)

        def quad_body(i, carry):
            k = 4 * i + 2
            phase(k, 2)
            phase(k + 1, 3)
            phase(k + 2, 0)
            phase(k + 3, 1)
            return carry

        lax.fori_loop(0, (nchunk - 6) // 4, quad_body, 0)
        phase(nchunk - 4, 2)
        phase(nchunk - 3, 3)
        phase(nchunk - 2, 0, idx=False)
        phase(nchunk - 1, 1, rows=False, idx=False)
        wait_scatter(sets[0])
        wait_scatter(sets[1])
        plsc.subcore_barrier()

        @pl.when(sid < NS - 1)
        def _out_main():
            pltpu.sync_copy(agg_sh.at[pl.ds(row0, stripe)],
                            out_hbm.at[cid, pl.ds(row0, stripe)])

        @pl.when(sid == NS - 1)
        def _out_last():
            pltpu.sync_copy(agg_sh.at[pl.ds(row0, last_rows)],
                            out_hbm.at[cid, pl.ds(row0, last_rows)])

    return edge_kernel(ei[0], ei[1], Df, Ds, Sf, Ss, zrows)


def _fuse_body(x_ref, a0_ref, a1_ref, surf_ref, wsl_ref, bsl_ref, wl_ref,
               num_ref, ss_ref, csl):
    @pl.when(pl.program_id(0) == 0)
    def _init():
        num_ref[...] = jnp.zeros_like(num_ref)
        ss_ref[...] = jnp.zeros_like(ss_ref)

    xb = x_ref[...]
    agg = a0_ref[...] + a1_ref[...]
    cad = xb.shape[1] - csl
    x_sl = xb[:, :csl] + agg[:, cad:]
    x_ad = xb[:, csl:] + agg[:, :cad]
    t = jnp.dot(x_sl.astype(jnp.bfloat16), wsl_ref[...].astype(jnp.bfloat16),
                preferred_element_type=jnp.float32) + bsl_ref[...]
    h = t * x_ad * surf_ref[...]
    h16 = h.astype(jnp.bfloat16).astype(jnp.float32)
    wl16 = wl_ref[...].astype(jnp.bfloat16).astype(jnp.float32)
    num_ref[...] += jnp.sum(h16 * wl16)
    ss_ref[...] += jnp.sum(surf_ref[...])


def _fuse(x, agg0, agg1, surf, W_lin_sl, b_lin_sl, wl_row, blk, csl):
    n, f = x.shape
    cad = f - csl
    grid = n // blk
    return pl.pallas_call(
        functools.partial(_fuse_body, csl=csl),
        grid=(grid,),
        in_specs=[
            pl.BlockSpec((blk, f), lambda i: (i, 0)),
            pl.BlockSpec((blk, f), lambda i: (i, 0)),
            pl.BlockSpec((blk, f), lambda i: (i, 0)),
            pl.BlockSpec((blk, 1), lambda i: (i, 0)),
            pl.BlockSpec((csl, cad), lambda i: (0, 0)),
            pl.BlockSpec((1, cad), lambda i: (0, 0)),
            pl.BlockSpec((1, cad), lambda i: (0, 0)),
        ],
        out_specs=[
            pl.BlockSpec((1, 1), lambda i: (0, 0)),
            pl.BlockSpec((1, 1), lambda i: (0, 0)),
        ],
        out_shape=[
            jax.ShapeDtypeStruct((1, 1), jnp.float32),
            jax.ShapeDtypeStruct((1, 1), jnp.float32),
        ],
    )(x, agg0, agg1, surf, W_lin_sl, b_lin_sl, wl_row)


def kernel(x, edge_index, surf_filter,
           Wf_sl, bf_sl, Ws_sl, bs_sl,
           Wf_ad, bf_ad, Ws_ad, bs_ad,
           W_lin_sl, b_lin_sl, W_lin, b_lin):
    n, f = x.shape
    e = edge_index.shape[1]
    csl = Wf_sl.shape[1]
    cad = Wf_ad.shape[1]

    z_sa = jnp.zeros((csl, cad), jnp.float32)
    z_as = jnp.zeros((cad, csl), jnp.float32)

    def pack(w_ad, w_sl):
        return jnp.concatenate([
            jnp.concatenate([z_sa, w_sl], axis=1),
            jnp.concatenate([w_ad, z_as], axis=1),
        ], axis=0)

    W_Df = -pack(Wf_ad[:cad], Wf_sl[:csl])
    W_Ds = pack(Ws_ad[:cad], Ws_sl[:csl])
    W_Sf = -pack(Wf_ad[cad:], Wf_sl[csl:])
    W_Ss = pack(Ws_ad[cad:], Ws_sl[csl:])
    b_f = -jnp.concatenate([bf_ad, bf_sl]).reshape(1, -1)
    b_s = jnp.concatenate([bs_ad, bs_sl]).reshape(1, -1)

    Df, Ds, Sf, Ss = _projections(x, W_Df, W_Ds, W_Sf, W_Ss, b_f, b_s, blk=2000)

    zrows = jnp.zeros((640, f), jnp.float32)
    agg = _edge_phase(edge_index, Df, Ds, Sf, Ss, zrows, n, e, chunk=40)

    num, ss = _fuse(x, agg[0], agg[1], surf_filter.reshape(-1, 1),
                    W_lin_sl, b_lin_sl.reshape(1, -1), W_lin.reshape(1, -1),
                    blk=2000, csl=csl)
    return (num[0, 0] + n * b_lin[0]) / ss[0, 0]
